# Initial kernel scaffold; baseline (speedup 1.0000x reference)
import jax
import jax.numpy as jnp
from jax.experimental import pallas as pl


def kernel(pred_logits, pred_obj, pred_boxes, target_sizes):
    raise NotImplementedError("write your pallas kernel here")


# speedup vs baseline: 6.0382x; 6.0382x over previous
import jax
import jax.numpy as jnp
from jax import lax
from jax.experimental import pallas as pl
from jax.experimental.pallas import tpu as pltpu

_C = 91
_K = 100
_NB = 1024
_FIRST_INVALID = 81
_NEG = float("-inf")
_BIG_I = 2**30


def _softplus(x):
    return jnp.maximum(x, 0.0) + jnp.log1p(jnp.exp(-jnp.abs(x)))


def _k1_body(l_ref, o_ref, unc_ref, s_ref, bs_ref):
    x = l_ref[0]
    lane = lax.broadcasted_iota(jnp.int32, x.shape, 1)
    alpha = jnp.where(lane >= _FIRST_INVALID, 1.0, _softplus(x) + 1.0)
    asum = jnp.sum(alpha, axis=1)
    amax = jnp.max(alpha, axis=1)
    obj = o_ref[0, 0]
    s = jnp.exp(-obj) / asum
    unc_ref[0, 0, :] = jnp.float32(_C) / asum
    s_ref[0, 0, :] = s
    bs_ref[0, 0, :] = s * amax


def _k1(pred_logits, obj3):
    B, N, C = pred_logits.shape
    grid = (B, (N + _NB - 1) // _NB)
    out_sd = jax.ShapeDtypeStruct((B, 1, N), jnp.float32)
    return pl.pallas_call(
        _k1_body,
        grid=grid,
        in_specs=[
            pl.BlockSpec((1, _NB, C), lambda b, j: (b, j, 0)),
            pl.BlockSpec((1, 1, _NB), lambda b, j: (b, 0, j)),
        ],
        out_specs=[
            pl.BlockSpec((1, 1, _NB), lambda b, j: (b, 0, j)),
            pl.BlockSpec((1, 1, _NB), lambda b, j: (b, 0, j)),
            pl.BlockSpec((1, 1, _NB), lambda b, j: (b, 0, j)),
        ],
        out_shape=[out_sd, out_sd, out_sd],
    )(pred_logits, obj3)


def _k2_body(bs_ref, s_ref, idx_ref, cs_ref, sc_ref):
    sc_ref[...] = bs_ref[...]
    col = lax.broadcasted_iota(jnp.int32, sc_ref.shape, 1)
    s = s_ref[...]
    for i in range(_K):
        v = sc_ref[...]
        m = jnp.max(v, axis=1, keepdims=True)
        tie = v == m
        nsel = jnp.min(jnp.where(tie, col, _BIG_I), axis=1, keepdims=True)
        chosen = col == nsel
        cs_val = jnp.sum(jnp.where(chosen, s, 0.0), axis=1, keepdims=True)
        sc_ref[...] = jnp.where(chosen, _NEG, v)
        idx_ref[:, pl.dslice(i, 1)] = nsel
        cs_ref[:, pl.dslice(i, 1)] = cs_val


def _k2(bscore, s):
    B, N = bscore.shape
    return pl.pallas_call(
        _k2_body,
        out_shape=[
            jax.ShapeDtypeStruct((B, _K), jnp.int32),
            jax.ShapeDtypeStruct((B, _K), jnp.float32),
        ],
        scratch_shapes=[pltpu.VMEM((B, N), jnp.float32)],
    )(bscore, s)


def _kg_body(idx_ref, l_ref, b_ref, gl_ref, gb_ref):
    gl_ref[...] = l_ref[...]
    gb_ref[...] = b_ref[...]


def _kgather(cand_idx, logits4, boxes4):
    B, N, _, C = logits4.shape
    grid_spec = pltpu.PrefetchScalarGridSpec(
        num_scalar_prefetch=1,
        grid=(B, _K),
        in_specs=[
            pl.BlockSpec((1, 1, 1, C), lambda b, i, idx: (b, idx[b, i], 0, 0)),
            pl.BlockSpec((1, 1, 1, 4), lambda b, i, idx: (b, idx[b, i], 0, 0)),
        ],
        out_specs=[
            pl.BlockSpec((1, 1, 1, C), lambda b, i, idx: (b, i, 0, 0)),
            pl.BlockSpec((1, 1, 1, 4), lambda b, i, idx: (b, i, 0, 0)),
        ],
    )
    return pl.pallas_call(
        _kg_body,
        grid_spec=grid_spec,
        out_shape=[
            jax.ShapeDtypeStruct((B, _K, 1, C), jnp.float32),
            jax.ShapeDtypeStruct((B, _K, 1, 4), jnp.float32),
        ],
    )(cand_idx, logits4, boxes4)


def _k3_body(gl_ref, gb_ref, ci_ref, cs_ref, ts_ref, sc_out, lb_out, bx_out):
    x = gl_ref[...]
    lane = lax.broadcasted_iota(jnp.int32, x.shape, 2)
    alpha = jnp.where(lane >= _FIRST_INVALID, 1.0, _softplus(x) + 1.0)
    s = cs_ref[...][:, :, None]
    val = s * alpha
    n = ci_ref[...][:, :, None]
    flat = n * _C + lane

    b = gb_ref[...]
    xc = b[:, :, 0:1]
    yc = b[:, :, 1:2]
    w = b[:, :, 2:3]
    h = b[:, :, 3:4]
    xyxy = jnp.concatenate(
        [xc - 0.5 * w, yc - 0.5 * h, xc + 0.5 * w, yc + 0.5 * h], axis=2)
    ts = ts_ref[...]
    img_h = ts[:, 0:1]
    img_w = ts[:, 1:2]
    scale = jnp.concatenate([img_w, img_h, img_w, img_h], axis=1)
    cb = xyxy * scale[:, None, :]

    v = val
    for i in range(_K):
        m2 = jnp.max(v, axis=2)
        m = jnp.max(m2, axis=1, keepdims=True)
        tie = v == m[:, :, None]
        fcand = jnp.where(tie, flat, _BIG_I)
        fsel = jnp.min(jnp.min(fcand, axis=2), axis=1, keepdims=True)
        chosen = flat == fsel[:, :, None]
        wrow = jnp.max(chosen.astype(jnp.float32), axis=2)
        box_i = jnp.sum(wrow[:, :, None] * cb, axis=1)
        sc_out[:, pl.dslice(i, 1)] = m
        lb_out[:, pl.dslice(i, 1)] = fsel % _C
        bx_out[:, pl.dslice(i, 1), :] = box_i[:, None, :]
        v = jnp.where(chosen, _NEG, v)


def _k3(gl, gb, cand_idx, cand_s, target_sizes):
    B = gl.shape[0]
    return pl.pallas_call(
        _k3_body,
        out_shape=[
            jax.ShapeDtypeStruct((B, _K), jnp.float32),
            jax.ShapeDtypeStruct((B, _K), jnp.int32),
            jax.ShapeDtypeStruct((B, _K, 4), jnp.float32),
        ],
    )(gl, gb, cand_idx, cand_s, target_sizes)


def kernel(pred_logits, pred_obj, pred_boxes, target_sizes):
    B, N, C = pred_logits.shape
    obj3 = pred_obj.reshape(B, 1, N)
    unc3, s3, bs3 = _k1(pred_logits, obj3)
    uncertainty = unc3.reshape(B, N, 1)
    cand_idx, cand_s = _k2(bs3.reshape(B, N), s3.reshape(B, N))
    gl4, gb4 = _kgather(cand_idx, pred_logits.reshape(B, N, 1, C),
                        pred_boxes.reshape(B, N, 1, 4))
    scores, labels, boxes = _k3(gl4.reshape(B, _K, C), gb4.reshape(B, _K, 4),
                                cand_idx, cand_s, target_sizes)
    return scores, labels, boxes, uncertainty
